# Initial kernel scaffold; baseline (speedup 1.0000x reference)
import jax
import jax.numpy as jnp
from jax.experimental import pallas as pl


def kernel(x, edge_index, edge_attr, en_w, en_b, ee_w, ee_b, Wq, Wk, Wv, att_b, bn_g, bn_b):
    raise NotImplementedError("write your pallas kernel here")


# speedup vs baseline: 1.2083x; 1.2083x over previous
import functools
import math

import jax
import jax.numpy as jnp
from jax import lax
from jax.experimental import pallas as pl
from jax.experimental.pallas import tpu as pltpu

N = 10000
E = 160000
HEADS = 4
F_IN = 92
F_EDGE = 41
NH = 64
NL = 3
D_CAT = NH + NH
D_OUT = HEADS * NH

EB = 2000


def _embed_kernel(x_ref, w_ref, b_ref, o_ref):
    t = jnp.dot(x_ref[...], w_ref[...], preferred_element_type=jnp.float32)
    o_ref[...] = jax.nn.softplus(t + b_ref[...])


def _edge_embed_kernel(a_ref, w_ref, b_ref, o_ref):
    t = jnp.dot(a_ref[...], w_ref[...], preferred_element_type=jnp.float32)
    t = t + b_ref[...]
    o_ref[...] = jnp.where(t >= 0, t, 0.2 * t)


def _qkv_kernel(xi_ref, xj_ref, ea_ref, wq_ref, wkv_ref, v_ref, es_ref):
    ea = ea_ref[...]
    xi = jnp.concatenate([xi_ref[...], ea], axis=1)
    xj = jnp.concatenate([xj_ref[...], ea], axis=1)
    q = jax.nn.softplus(
        jnp.dot(xi, wq_ref[...], preferred_element_type=jnp.float32))
    kv = jnp.dot(xj, wkv_ref[...], preferred_element_type=jnp.float32)
    k = jax.nn.softplus(kv[:, :D_OUT])
    v_ref[...] = jax.nn.softplus(kv[:, D_OUT:])
    qk = q * k
    parts = [
        jnp.sum(qk[:, h * NH:(h + 1) * NH], axis=1, keepdims=True)
        for h in range(HEADS)
    ]
    s = jnp.concatenate(parts, axis=1) * (1.0 / math.sqrt(NH))
    es_ref[...] = jnp.exp(s)


def _edge_out_kernel(v_ref, es_ref, den_ref, xi_ref, b_ref, y_ref):
    al = es_ref[...] / (den_ref[...] + 1e-16)
    v = v_ref[...]
    out = al[:, 0:1] * v[:, 0:NH]
    for h in range(1, HEADS):
        out = out + al[:, h:h + 1] * v[:, h * NH:(h + 1) * NH]
    out = out * (1.0 / HEADS) + b_ref[...]
    mu = jnp.mean(out, axis=1, keepdims=True)
    var = jnp.mean((out - mu) ** 2, axis=1, keepdims=True)
    out = (out - mu) / jnp.sqrt(var + 1e-5)
    y_ref[...] = jax.nn.softplus(out + xi_ref[...])


def _bn_kernel(pa_ref, pb_ref, g_ref, b_ref, o_ref):
    out = pa_ref[...] + pb_ref[...]
    mu = jnp.mean(out, axis=0, keepdims=True)
    var = jnp.mean((out - mu) ** 2, axis=0, keepdims=True)
    o_ref[...] = g_ref[...] * (out - mu) / jnp.sqrt(var + 1e-5) + b_ref[...]


def _node_embed(x, en_w, en_b):
    return pl.pallas_call(
        _embed_kernel,
        out_shape=jax.ShapeDtypeStruct((N, NH), jnp.float32),
    )(x, en_w, en_b.reshape(1, NH))


def _edge_embed(edge_attr, ee_w, ee_b):
    grid = 20
    rb = E // grid
    return pl.pallas_call(
        _edge_embed_kernel,
        grid=(grid,),
        in_specs=[
            pl.BlockSpec((rb, F_EDGE), lambda i: (i, 0)),
            pl.BlockSpec((F_EDGE, NH), lambda i: (0, 0)),
            pl.BlockSpec((1, NH), lambda i: (0, 0)),
        ],
        out_specs=pl.BlockSpec((rb, NH), lambda i: (i, 0)),
        out_shape=jax.ShapeDtypeStruct((E, NH), jnp.float32),
    )(edge_attr, ee_w, ee_b.reshape(1, NH))


def _qkv_scores(x_i, x_j, ea, wq, wkv):
    grid = E // EB
    return pl.pallas_call(
        _qkv_kernel,
        grid=(grid,),
        in_specs=[
            pl.BlockSpec((EB, NH), lambda i: (i, 0)),
            pl.BlockSpec((EB, NH), lambda i: (i, 0)),
            pl.BlockSpec((EB, NH), lambda i: (i, 0)),
            pl.BlockSpec((D_CAT, D_OUT), lambda i: (0, 0)),
            pl.BlockSpec((D_CAT, 2 * D_OUT), lambda i: (0, 0)),
        ],
        out_specs=[
            pl.BlockSpec((EB, D_OUT), lambda i: (i, 0)),
            pl.BlockSpec((EB, HEADS), lambda i: (i, 0)),
        ],
        out_shape=[
            jax.ShapeDtypeStruct((E, D_OUT), jnp.float32),
            jax.ShapeDtypeStruct((E, HEADS), jnp.float32),
        ],
    )(x_i, x_j, ea, wq, wkv)


def _edge_out(v, es, den_e, x_i, att_b):
    grid = E // EB
    return pl.pallas_call(
        _edge_out_kernel,
        grid=(grid,),
        in_specs=[
            pl.BlockSpec((EB, D_OUT), lambda i: (i, 0)),
            pl.BlockSpec((EB, HEADS), lambda i: (i, 0)),
            pl.BlockSpec((EB, HEADS), lambda i: (i, 0)),
            pl.BlockSpec((EB, NH), lambda i: (i, 0)),
            pl.BlockSpec((1, NH), lambda i: (0, 0)),
        ],
        out_specs=pl.BlockSpec((EB, NH), lambda i: (i, 0)),
        out_shape=jax.ShapeDtypeStruct((E, NH), jnp.float32),
    )(v, es, den_e, x_i, att_b.reshape(1, NH))


def _batchnorm(pa, pb, g, b):
    return pl.pallas_call(
        _bn_kernel,
        out_shape=jax.ShapeDtypeStruct((N, NH), jnp.float32),
    )(pa, pb, g.reshape(1, NH), b.reshape(1, NH))


def _gather_rows(h, seg_i, idx_j):
    return h[seg_i], h[idx_j]


def _denom(es, seg_i):
    den = jax.ops.segment_sum(es, seg_i, num_segments=N)
    return den[seg_i]


def _segsum(y, seg_i):
    agg = jax.ops.segment_sum(y, seg_i, num_segments=N)
    return agg, jnp.zeros_like(agg)


def kernel(x, edge_index, edge_attr, en_w, en_b, ee_w, ee_b, Wq, Wk, Wv,
           att_b, bn_g, bn_b):
    seg_i = edge_index[0]
    idx_j = edge_index[1]
    h = _node_embed(x, en_w, en_b)
    ea = _edge_embed(edge_attr, ee_w, ee_b)
    for l in range(NL):
        wq = Wq[l]
        wkv = jnp.concatenate([Wk[l], Wv[l]], axis=1)
        x_i, x_j = _gather_rows(h, seg_i, idx_j)
        v, es = _qkv_scores(x_i, x_j, ea, wq, wkv)
        den_e = _denom(es, seg_i)
        y = _edge_out(v, es, den_e, x_i, att_b[l])
        pa, pb = _segsum(y, seg_i)
        h = _batchnorm(pa, pb, bn_g[l], bn_b[l])
    return h


# speedup vs baseline: 1.9904x; 1.6473x over previous
import functools
import math

import jax
import jax.numpy as jnp
from jax import lax
from jax.experimental import pallas as pl
from jax.experimental.pallas import tpu as pltpu
from jax.experimental.pallas import tpu_sc as plsc

N = 10000
E = 160000
HEADS = 4
F_IN = 92
F_EDGE = 41
NH = 64
NL = 3
D_CAT = NH + NH
D_OUT = HEADS * NH

EB = 2000


def _embed_kernel(x_ref, w_ref, b_ref, o_ref):
    t = jnp.dot(x_ref[...], w_ref[...], preferred_element_type=jnp.float32)
    o_ref[...] = jax.nn.softplus(t + b_ref[...])


def _edge_embed_kernel(a_ref, w_ref, b_ref, o_ref):
    t = jnp.dot(a_ref[...], w_ref[...], preferred_element_type=jnp.float32)
    t = t + b_ref[...]
    o_ref[...] = jnp.where(t >= 0, t, 0.2 * t)


def _qkv_kernel(xi_ref, xj_ref, ea_ref, wq_ref, wkv_ref, v_ref, es_ref):
    ea = ea_ref[...]
    xi = jnp.concatenate([xi_ref[...], ea], axis=1)
    xj = jnp.concatenate([xj_ref[...], ea], axis=1)
    q = jax.nn.softplus(
        jnp.dot(xi, wq_ref[...], preferred_element_type=jnp.float32))
    kv = jnp.dot(xj, wkv_ref[...], preferred_element_type=jnp.float32)
    k = jax.nn.softplus(kv[:, :D_OUT])
    v_ref[...] = jax.nn.softplus(kv[:, D_OUT:])
    qk = q * k
    parts = [
        jnp.sum(qk[:, h * NH:(h + 1) * NH], axis=1, keepdims=True)
        for h in range(HEADS)
    ]
    s = jnp.concatenate(parts, axis=1) * (1.0 / math.sqrt(NH))
    es = jnp.exp(s)
    es_ref[...] = jnp.concatenate(
        [es, jnp.zeros((es.shape[0], DENW - HEADS), jnp.float32)], axis=1)


def _edge_out_kernel(v_ref, es_ref, den_ref, xi_ref, b_ref, y_ref):
    al = es_ref[:, :HEADS] / (den_ref[:, :HEADS] + 1e-16)
    v = v_ref[...]
    out = al[:, 0:1] * v[:, 0:NH]
    for h in range(1, HEADS):
        out = out + al[:, h:h + 1] * v[:, h * NH:(h + 1) * NH]
    out = out * (1.0 / HEADS) + b_ref[...]
    mu = jnp.mean(out, axis=1, keepdims=True)
    var = jnp.mean((out - mu) ** 2, axis=1, keepdims=True)
    out = (out - mu) / jnp.sqrt(var + 1e-5)
    y_ref[...] = jax.nn.softplus(out + xi_ref[...])


def _bn_kernel(pa_ref, pb_ref, g_ref, b_ref, o_ref):
    out = pa_ref[...] + pb_ref[...]
    mu = jnp.mean(out, axis=0, keepdims=True)
    var = jnp.mean((out - mu) ** 2, axis=0, keepdims=True)
    o_ref[...] = g_ref[...] * (out - mu) / jnp.sqrt(var + 1e-5) + b_ref[...]


def _node_embed(x, en_w, en_b):
    return pl.pallas_call(
        _embed_kernel,
        out_shape=jax.ShapeDtypeStruct((N, NH), jnp.float32),
    )(x, en_w, en_b.reshape(1, NH))


def _edge_embed(edge_attr, ee_w, ee_b):
    grid = 20
    rb = E // grid
    return pl.pallas_call(
        _edge_embed_kernel,
        grid=(grid,),
        in_specs=[
            pl.BlockSpec((rb, F_EDGE), lambda i: (i, 0)),
            pl.BlockSpec((F_EDGE, NH), lambda i: (0, 0)),
            pl.BlockSpec((1, NH), lambda i: (0, 0)),
        ],
        out_specs=pl.BlockSpec((rb, NH), lambda i: (i, 0)),
        out_shape=jax.ShapeDtypeStruct((E, NH), jnp.float32),
    )(edge_attr, ee_w, ee_b.reshape(1, NH))


def _qkv_scores(x_i, x_j, ea, wq, wkv):
    grid = E // EB
    return pl.pallas_call(
        _qkv_kernel,
        grid=(grid,),
        in_specs=[
            pl.BlockSpec((EB, NH), lambda i: (i, 0)),
            pl.BlockSpec((EB, NH), lambda i: (i, 0)),
            pl.BlockSpec((EB, NH), lambda i: (i, 0)),
            pl.BlockSpec((D_CAT, D_OUT), lambda i: (0, 0)),
            pl.BlockSpec((D_CAT, 2 * D_OUT), lambda i: (0, 0)),
        ],
        out_specs=[
            pl.BlockSpec((EB, D_OUT), lambda i: (i, 0)),
            pl.BlockSpec((EB, DENW), lambda i: (i, 0)),
        ],
        out_shape=[
            jax.ShapeDtypeStruct((E, D_OUT), jnp.float32),
            jax.ShapeDtypeStruct((E, DENW), jnp.float32),
        ],
    )(x_i, x_j, ea, wq, wkv)


def _edge_out(v, es, den_e, x_i, att_b):
    grid = E // EB
    return pl.pallas_call(
        _edge_out_kernel,
        grid=(grid,),
        in_specs=[
            pl.BlockSpec((EB, D_OUT), lambda i: (i, 0)),
            pl.BlockSpec((EB, DENW), lambda i: (i, 0)),
            pl.BlockSpec((EB, DENW), lambda i: (i, 0)),
            pl.BlockSpec((EB, NH), lambda i: (i, 0)),
            pl.BlockSpec((1, NH), lambda i: (0, 0)),
        ],
        out_specs=pl.BlockSpec((EB, NH), lambda i: (i, 0)),
        out_shape=jax.ShapeDtypeStruct((E, NH), jnp.float32),
    )(v, es, den_e, x_i, att_b.reshape(1, NH))


def _batchnorm(pa, pb, g, b):
    return pl.pallas_call(
        _bn_kernel,
        out_shape=jax.ShapeDtypeStruct((N, NH), jnp.float32),
    )(pa, pb, g.reshape(1, NH), b.reshape(1, NH))


_NC = 2
_NS = 16
_NW = _NC * _NS
_GB = 128
_NB = E // _GB
DENW = 16

_MESH = plsc.VectorSubcoreMesh(core_axis_name="c", subcore_axis_name="s")
_SC_PARAMS = pltpu.CompilerParams(use_tc_tiling_on_sc=False)


def _sc_gather(h, seg_i, idx_j):
    @functools.partial(
        pl.kernel,
        out_type=[
            jax.ShapeDtypeStruct((E, NH), jnp.float32),
            jax.ShapeDtypeStruct((E, NH), jnp.float32),
        ],
        mesh=_MESH,
        compiler_params=_SC_PARAMS,
        scratch_types=[
            pltpu.VMEM((_GB,), jnp.int32),
            pltpu.VMEM((_GB, NH), jnp.float32),
            pltpu.VMEM((_GB,), jnp.int32),
            pltpu.VMEM((_GB, NH), jnp.float32),
            pltpu.SemaphoreType.DMA,
            pltpu.SemaphoreType.DMA,
        ],
    )
    def k(h_hbm, si_hbm, sj_hbm, oi_hbm, oj_hbm, ia, ra, ib, rb, sa, sb):
        wid = lax.axis_index("s") * _NC + lax.axis_index("c")
        nb = _NB // _NW + jnp.where(wid < _NB % _NW, 1, 0)

        def body(i, carry):
            off = (wid + i * _NW) * _GB
            pltpu.sync_copy(si_hbm.at[pl.ds(off, _GB)], ia)
            pltpu.sync_copy(sj_hbm.at[pl.ds(off, _GB)], ib)
            ca = pltpu.async_copy(h_hbm.at[ia], ra, sa)
            cb = pltpu.async_copy(h_hbm.at[ib], rb, sb)
            ca.wait()
            cb.wait()
            pltpu.sync_copy(ra, oi_hbm.at[pl.ds(off, _GB)])
            pltpu.sync_copy(rb, oj_hbm.at[pl.ds(off, _GB)])
            return carry

        lax.fori_loop(0, nb, body, 0)

    return k(h, seg_i, idx_j)


def _sc_denom(es, seg_i, zeros_nw):
    @functools.partial(
        pl.kernel,
        out_type=jax.ShapeDtypeStruct((E, DENW), jnp.float32),
        mesh=_MESH,
        compiler_params=_SC_PARAMS,
        scratch_types=[
            pltpu.VMEM_SHARED((N, DENW), jnp.float32),
            pltpu.VMEM((_GB,), jnp.int32),
            pltpu.VMEM((_GB, DENW), jnp.float32),
            pltpu.SemaphoreType.DMA,
        ],
    )
    def k(es_hbm, si_hbm, z_hbm, out_hbm, table, ib, vb, sem):
        s = lax.axis_index("s")
        c = lax.axis_index("c")
        wid = s * _NC + c

        stripe = N // _NS
        pltpu.sync_copy(z_hbm.at[pl.ds(s * stripe, stripe)],
                        table.at[pl.ds(s * stripe, stripe)])
        plsc.subcore_barrier()

        nb = _NB // _NS + jnp.where(s < _NB % _NS, 1, 0)

        def body(i, carry):
            off = (s + i * _NS) * _GB
            pltpu.sync_copy(si_hbm.at[pl.ds(off, _GB)], ib)
            pltpu.sync_copy(es_hbm.at[pl.ds(off, _GB)], vb)
            pltpu.sync_copy(vb, table.at[ib], add=True)
            return carry

        lax.fori_loop(0, nb, body, 0)
        plsc.subcore_barrier()

        nb2 = _NB // _NW + jnp.where(wid < _NB % _NW, 1, 0)

        def body2(i, carry):
            off = (wid + i * _NW) * _GB
            pltpu.sync_copy(si_hbm.at[pl.ds(off, _GB)], ib)
            pltpu.async_copy(table.at[ib], vb, sem).wait()
            pltpu.sync_copy(vb, out_hbm.at[pl.ds(off, _GB)])
            return carry

        lax.fori_loop(0, nb2, body2, 0)

    return k(es, seg_i, zeros_nw)


def _sc_segsum(y, seg_i, zeros_n):
    @functools.partial(
        pl.kernel,
        out_type=jax.ShapeDtypeStruct((2 * N, NH), jnp.float32),
        mesh=_MESH,
        compiler_params=_SC_PARAMS,
        scratch_types=[
            pltpu.VMEM_SHARED((N, NH), jnp.float32),
            pltpu.VMEM((_GB,), jnp.int32),
            pltpu.VMEM((_GB, NH), jnp.float32),
        ],
    )
    def k(y_hbm, si_hbm, z_hbm, out_hbm, table, ib, vb):
        s = lax.axis_index("s")
        c = lax.axis_index("c")

        stripe = N // _NS
        pltpu.sync_copy(z_hbm.at[pl.ds(s * stripe, stripe)],
                        table.at[pl.ds(s * stripe, stripe)])
        plsc.subcore_barrier()

        hb = _NB // _NC
        nb = hb // _NS + jnp.where(s < hb % _NS, 1, 0)

        def body(i, carry):
            off = (c * hb + s + i * _NS) * _GB
            pltpu.sync_copy(si_hbm.at[pl.ds(off, _GB)], ib)
            pltpu.sync_copy(y_hbm.at[pl.ds(off, _GB)], vb)
            pltpu.sync_copy(vb, table.at[ib], add=True)
            return carry

        lax.fori_loop(0, nb, body, 0)
        plsc.subcore_barrier()

        pltpu.sync_copy(table.at[pl.ds(s * stripe, stripe)],
                        out_hbm.at[pl.ds(c * N + s * stripe, stripe)])

    return k(y, seg_i, zeros_n)


def kernel(x, edge_index, edge_attr, en_w, en_b, ee_w, ee_b, Wq, Wk, Wv,
           att_b, bn_g, bn_b):
    seg_i = edge_index[0]
    idx_j = edge_index[1]
    zeros_nw = jnp.zeros((N, DENW), jnp.float32)
    zeros_n = jnp.zeros((N, NH), jnp.float32)
    h = _node_embed(x, en_w, en_b)
    ea = _edge_embed(edge_attr, ee_w, ee_b)
    for l in range(NL):
        wq = Wq[l]
        wkv = jnp.concatenate([Wk[l], Wv[l]], axis=1)
        x_i, x_j = _sc_gather(h, seg_i, idx_j)
        v, es = _qkv_scores(x_i, x_j, ea, wq, wkv)
        den_e = _sc_denom(es, seg_i, zeros_nw)
        y = _edge_out(v, es, den_e, x_i, att_b[l])
        agg2 = _sc_segsum(y, seg_i, zeros_n)
        h = _batchnorm(agg2[:N], agg2[N:], bn_g[l], bn_b[l])
    return h
